# Initial kernel scaffold; baseline (speedup 1.0000x reference)
import jax
import jax.numpy as jnp
from jax.experimental import pallas as pl


def kernel(pressure, temperature, w1, b1, w2, b2, p_emb, t_emb):
    raise NotImplementedError("write your pallas kernel here")


# speedup vs baseline: 4.6883x; 4.6883x over previous
import jax
import jax.numpy as jnp
from jax import lax
from jax.experimental import pallas as pl
from jax.experimental.pallas import tpu as pltpu

B = 16384
H = 128
BINS = 32
_ROWS = 2048


def _body(s_ref, wp_ref, pe_ref, te_ref, p_ref, t_ref, o_ref):
    p = p_ref[...]
    t = t_ref[...]
    pc = jnp.clip(p, 0.0, 1.0)
    tc = jnp.clip(t, 0.0, 1.0)

    h0 = jnp.maximum(pc * s_ref[0] + tc * s_ref[2] + s_ref[4], 0.0)
    h1 = jnp.maximum(pc * s_ref[1] + tc * s_ref[3] + s_ref[5], 0.0)
    proj = h0 * wp_ref[0:1, :] + h1 * wp_ref[1:2, :] + wp_ref[2:3, :]

    ki = lax.broadcasted_iota(jnp.int32, (1, BINS), 1)
    lo = ki.astype(jnp.float32) * (1.0 / BINS)
    hi = jnp.where(ki == BINS - 1, jnp.inf, lo + 1.0 / BINS)
    oh_p = ((pc >= lo) & (pc < hi)).astype(jnp.float32)
    oh_t = ((tc >= lo) & (tc < hi)).astype(jnp.float32)

    pe = jnp.dot(oh_p, pe_ref[...], preferred_element_type=jnp.float32,
                 precision=lax.Precision.HIGHEST)
    te = jnp.dot(oh_t, te_ref[...], preferred_element_type=jnp.float32,
                 precision=lax.Precision.HIGHEST)

    o_ref[:, 0:H] = proj
    o_ref[:, H : 2 * H] = pe
    o_ref[:, 2 * H : 3 * H] = te


def kernel(pressure, temperature, w1, b1, w2, b2, p_emb, t_emb):
    s = jnp.concatenate([w1.reshape(-1), b1.reshape(-1)])
    wp = jnp.zeros((8, H), jnp.float32).at[0:2].set(w2).at[2].set(b2)
    p2 = pressure[:, None]
    t2 = temperature[:, None]

    grid = (B // _ROWS,)
    out = pl.pallas_call(
        _body,
        grid=grid,
        in_specs=[
            pl.BlockSpec(memory_space=pltpu.SMEM),
            pl.BlockSpec((8, H), lambda i: (0, 0)),
            pl.BlockSpec((BINS, H), lambda i: (0, 0)),
            pl.BlockSpec((BINS, H), lambda i: (0, 0)),
            pl.BlockSpec((_ROWS, 1), lambda i: (i, 0)),
            pl.BlockSpec((_ROWS, 1), lambda i: (i, 0)),
        ],
        out_specs=pl.BlockSpec((_ROWS, 3 * H), lambda i: (i, 0)),
        out_shape=jax.ShapeDtypeStruct((B, 3 * H), jnp.float32),
    )(s, wp, p_emb, t_emb, p2, t2)
    return out
